# speedup vs baseline: 1.0233x; 1.0233x over previous
import functools

import jax
import jax.numpy as jnp
from jax import lax
from jax.experimental import pallas as pl
from jax.experimental.pallas import tpu as pltpu
from jax.experimental.pallas import tpu_sc as plsc

_LANES = 16


@functools.cache
def _build(BT: int, V: int, TPOS: int, D: int, C: int):
    info = plsc.get_sparse_core_info()
    nw = info.num_cores * info.num_subcores
    rows_per_w = BT // nw
    n_chunks = rows_per_w // C
    mesh = plsc.VectorSubcoreMesh(core_axis_name="c", subcore_axis_name="s")

    @functools.partial(
        pl.kernel,
        mesh=mesh,
        out_type=jax.ShapeDtypeStruct((BT, D), jnp.float32),
        scratch_types=[
            pltpu.VMEM((C,), jnp.int32),
            pltpu.VMEM((C, D), jnp.float32),
            pltpu.VMEM((C, D), jnp.float32),
            pltpu.SemaphoreType.DMA,
        ],
    )
    def emb_kernel(idx_hbm, wte_hbm, wpe_hbm, out_hbm, idx_v, rows_v, wpe_v, sem):
        wid = lax.axis_index("s") * info.num_cores + lax.axis_index("c")
        base = wid * rows_per_w
        t_base = lax.rem(base, TPOS)
        for c in range(n_chunks):
            off = base + c * C
            pltpu.sync_copy(idx_hbm.at[pl.ds(off, C)], idx_v)
            gather = pltpu.async_copy(wte_hbm.at[idx_v], rows_v, sem)
            pltpu.sync_copy(wpe_hbm.at[pl.ds(t_base + c * C, C)], wpe_v)
            gather.wait()

            def add_row(i, carry):
                for j in range(D // _LANES):
                    sl = pl.ds(j * _LANES, _LANES)
                    rows_v[i, sl] = rows_v[i, sl] + wpe_v[i, sl]
                return carry

            lax.fori_loop(0, C, add_row, 0)
            pltpu.sync_copy(rows_v, out_hbm.at[pl.ds(off, C)])

    return emb_kernel


def kernel(idx, wte, wpe):
    b, t = idx.shape
    v, d = wte.shape
    tpos = wpe.shape[0]
    idx_flat = idx.reshape(b * t).astype(jnp.int32)
    out = _build(b * t, v, tpos, d, 64)(idx_flat, wte, wpe)
    return out.reshape(b, t, d)
